# Initial kernel scaffold; baseline (speedup 1.0000x reference)
import jax
import jax.numpy as jnp
from jax.experimental import pallas as pl


def kernel(x, edge_index, edge_attr, batch, W_vert, W_edge, W_elin, W_conv, b_conv, W_res, W_h1, b_h1, W_h2, b_h2):
    raise NotImplementedError("write your pallas kernel here")


# speedup vs baseline: 1.3527x; 1.3527x over previous
import functools
import math

import jax
import jax.numpy as jnp
from jax import lax
from jax.experimental import pallas as pl
from jax.experimental.pallas import tpu as pltpu
from jax.experimental.pallas import tpu_sc as plsc

N_NODES = 50000
N_EDGES = 800000
NG = 64
H = 32
NL = 5
WIDTHS = [1, 1, 2, 3, 5, 7]
CIN = [w * H for w in WIDTHS[:-1]]
COUT = [w * H for w in WIDTHS[1:]]
INV_SQRT2 = 1.0 / math.sqrt(2.0)
EPS1 = 1.0 + 1e-05

NC = 2
NS = 16
LANES = 16

EPT = N_EDGES // NS
BE_SC = 2000
NB_SC = EPT // BE_SC
GCH = 80
NGC = BE_SC // GCH
N_PAD = 50048
RPT = N_PAD // NS
ZROWS = 184


@functools.lru_cache(maxsize=None)
def _make_sc_aggr(C):
    mesh = plsc.VectorSubcoreMesh(core_axis_name="c", subcore_axis_name="s")

    @functools.partial(
        pl.kernel,
        out_type=jax.ShapeDtypeStruct((C * N_PAD, LANES), jnp.float32),
        mesh=mesh,
        scratch_types=[
            pltpu.VMEM((NGC, GCH), jnp.int32),
            pltpu.VMEM((NGC, GCH), jnp.int32),
            pltpu.VMEM((NGC, GCH), jnp.int32),
            pltpu.VMEM((BE_SC, LANES), jnp.float32),
            pltpu.VMEM((BE_SC, LANES), jnp.float32),
            pltpu.VMEM((ZROWS, LANES), jnp.float32),
            pltpu.VMEM_SHARED((N_PAD, LANES), jnp.float32),
            pltpu.SemaphoreType.DMA,
        ],
        compiler_params=pltpu.CompilerParams(use_tc_tiling_on_sc=False),
    )
    def sc_aggr(src_hbm, dst_hbm, h_hbm, e_hbm, out_hbm,
                srcv, dstv, idxv, hrows, erows, zrows, acc, sem):
        core = lax.axis_index("c")
        tile = lax.axis_index("s")

        def zfill(r, carry):
            zrows[r] = jnp.zeros((LANES,), jnp.float32)
            return carry
        lax.fori_loop(0, ZROWS, zfill, 0)

        def chunk_body(j, carry):
            c = j * NC + core

            for k in range(RPT // ZROWS):
                pltpu.sync_copy(zrows, acc.at[pl.ds(tile * RPT + k * ZROWS, ZROWS)])
            plsc.subcore_barrier()

            def batch_body(b, inner):
                e0 = tile * EPT + b * BE_SC
                for g in range(NGC):
                    pltpu.sync_copy(src_hbm.at[pl.ds(e0 + g * GCH, GCH)], srcv.at[g])
                    pltpu.sync_copy(dst_hbm.at[pl.ds(e0 + g * GCH, GCH)], dstv.at[g])
                pltpu.sync_copy(e_hbm.at[pl.ds(c * N_EDGES + e0, BE_SC)], erows)

                cbase = jnp.full((LANES,), c * N_NODES, jnp.int32)

                def idx_body(g, carry2):
                    for k in range(GCH // LANES):
                        sl = pl.ds(k * LANES, LANES)
                        idxv[g, sl] = srcv[g, sl] + cbase
                    return carry2
                lax.fori_loop(0, NGC, idx_body, 0)

                descs = [
                    pltpu.async_copy(h_hbm.at[idxv.at[g]],
                                     hrows.at[pl.ds(g * GCH, GCH)], sem)
                    for g in range(NGC)
                ]
                for d in descs:
                    d.wait()

                def relu_body(r, carry2):
                    for u in range(8):
                        row = r * 8 + u
                        erows[row] = jnp.maximum(hrows[row] + erows[row], 0.0)
                    return carry2
                lax.fori_loop(0, BE_SC // 8, relu_body, 0)

                for g in range(NGC):
                    pltpu.sync_copy(erows.at[pl.ds(g * GCH, GCH)],
                                    acc.at[dstv.at[g]], add=True)
                return inner
            lax.fori_loop(0, NB_SC, batch_body, 0)
            plsc.subcore_barrier()

            pltpu.sync_copy(acc.at[pl.ds(tile * RPT, RPT)],
                            out_hbm.at[pl.ds(c * N_PAD + tile * RPT, RPT)])
            plsc.subcore_barrier()
            return carry
        lax.fori_loop(0, C // NC, chunk_body, 0)

    return sc_aggr


BN_INIT = 2000


def _tc_init_body(x_ref, wv_ref, out_ref):
    h = jnp.dot(x_ref[...], wv_ref[...], preferred_element_type=jnp.float32)
    for cc in range(H // LANES):
        out_ref[cc] = h[:, cc * LANES:(cc + 1) * LANES]


def _tc_init(x, w_vert):
    return pl.pallas_call(
        _tc_init_body,
        grid=(N_NODES // BN_INIT,),
        in_specs=[
            pl.BlockSpec((BN_INIT, 13), lambda n: (n, 0)),
            pl.BlockSpec((13, H), lambda n: (0, 0)),
        ],
        out_specs=pl.BlockSpec((H // LANES, BN_INIT, LANES), lambda n: (0, n, 0)),
        out_shape=jax.ShapeDtypeStruct((H // LANES, N_NODES, LANES), jnp.float32),
    )(x, w_vert)


BE_TC = 2000


def _tc_edge_body(ea_ref, we_ref, *rest):
    elin_refs = rest[:NL]
    out_refs = rest[NL:]
    t = jnp.dot(ea_ref[...], we_ref[...], preferred_element_type=jnp.float32)
    for i in range(NL):
        f = jnp.dot(t, elin_refs[i][...], preferred_element_type=jnp.float32)
        for cc in range(CIN[i] // LANES):
            out_refs[i][cc] = f[:, cc * LANES:(cc + 1) * LANES]


def _tc_edges(edge_attr, w_edge, w_elin):
    return pl.pallas_call(
        _tc_edge_body,
        grid=(N_EDGES // BE_TC,),
        in_specs=[
            pl.BlockSpec((BE_TC, 4), lambda n: (n, 0)),
            pl.BlockSpec((4, H), lambda n: (0, 0)),
        ] + [
            pl.BlockSpec((H, CIN[i]), lambda n: (0, 0)) for i in range(NL)
        ],
        out_specs=[
            pl.BlockSpec((CIN[i] // LANES, BE_TC, LANES), lambda n: (0, n, 0))
            for i in range(NL)
        ],
        out_shape=[
            jax.ShapeDtypeStruct((CIN[i] // LANES, N_EDGES, LANES), jnp.float32)
            for i in range(NL)
        ],
    )(edge_attr, w_edge, *w_elin)


BN_NODE = 1000


def _make_tc_node(i):
    ci_c = CIN[i] // LANES
    co_c = COUT[i] // LANES

    def body(h_ref, a_ref, wc_ref, bc_ref, wr_ref, out_ref):
        hcat = jnp.concatenate([h_ref[cc] for cc in range(ci_c)], axis=1)
        acat = jnp.concatenate([a_ref[cc] for cc in range(ci_c)], axis=1)
        z = EPS1 * hcat + acat
        out = jnp.maximum(
            jnp.dot(z, wc_ref[...], preferred_element_type=jnp.float32) + bc_ref[...],
            0.0)
        hn = (out + jnp.dot(hcat, wr_ref[...], preferred_element_type=jnp.float32)
              ) * INV_SQRT2
        for cc in range(co_c):
            out_ref[cc] = hn[:, cc * LANES:(cc + 1) * LANES]

    def run(h3, aggr3, w_conv, b_conv, w_res):
        return pl.pallas_call(
            body,
            grid=(N_NODES // BN_NODE,),
            in_specs=[
                pl.BlockSpec((ci_c, BN_NODE, LANES), lambda n: (0, n, 0)),
                pl.BlockSpec((ci_c, BN_NODE, LANES), lambda n: (0, n, 0)),
                pl.BlockSpec((CIN[i], COUT[i]), lambda n: (0, 0)),
                pl.BlockSpec((1, COUT[i]), lambda n: (0, 0)),
                pl.BlockSpec((CIN[i], COUT[i]), lambda n: (0, 0)),
            ],
            out_specs=pl.BlockSpec((co_c, BN_NODE, LANES), lambda n: (0, n, 0)),
            out_shape=jax.ShapeDtypeStruct((co_c, N_NODES, LANES), jnp.float32),
        )(h3, aggr3, w_conv, b_conv, w_res)
    return run


_TC_NODE = [_make_tc_node(i) for i in range(NL)]

BN_POOL = 1000
G_POOL = N_NODES // BN_POOL
CF = COUT[-1]
CF_C = CF // LANES


def _erf(x):
    a1, a2, a3, a4, a5 = (0.254829592, -0.284496736, 1.421413741,
                          -1.453152027, 1.061405429)
    p = 0.3275911
    s = jnp.sign(x)
    ax = jnp.abs(x)
    t = 1.0 / (1.0 + p * ax)
    poly = ((((a5 * t + a4) * t + a3) * t + a2) * t + a1) * t
    return s * (1.0 - poly * jnp.exp(-ax * ax))


def _tc_pool_head_body(h_ref, b_ref, wh1_ref, bh1_ref, wh2_ref, bh2_ref,
                       out_ref, acc_ref):
    n = pl.program_id(0)
    bid = b_ref[0]
    onehot_t = (lax.broadcasted_iota(jnp.int32, (NG, BN_POOL), 0)
                == jnp.broadcast_to(bid, (NG, BN_POOL))).astype(jnp.float32)
    hcat = jnp.concatenate(
        [h_ref[cc] for cc in range(CF_C)]
        + [jnp.ones((BN_POOL, LANES), jnp.float32)], axis=1)
    part = jnp.dot(onehot_t, hcat, preferred_element_type=jnp.float32)

    @pl.when(n == 0)
    def _init():
        acc_ref[...] = part

    @pl.when(n > 0)
    def _accum():
        acc_ref[...] = acc_ref[...] + part

    @pl.when(n == G_POOL - 1)
    def _head():
        sums = acc_ref[:, :CF]
        cnt = jnp.maximum(acc_ref[:, CF:CF + 1], 1.0)
        pooled = sums / cnt
        g1 = jnp.dot(pooled, wh1_ref[...],
                     preferred_element_type=jnp.float32) + bh1_ref[...]
        ge = g1 * 0.5 * (1.0 + _erf(g1 * INV_SQRT2))
        out_ref[...] = jnp.dot(ge, wh2_ref[...],
                               preferred_element_type=jnp.float32) + bh2_ref[...]


def _tc_pool_head(h3, batch3, w_h1, b_h1, w_h2, b_h2):
    return pl.pallas_call(
        _tc_pool_head_body,
        grid=(G_POOL,),
        in_specs=[
            pl.BlockSpec((CF_C, BN_POOL, LANES), lambda n: (0, n, 0)),
            pl.BlockSpec((1, 1, BN_POOL), lambda n: (n, 0, 0)),
            pl.BlockSpec((CF, 512), lambda n: (0, 0)),
            pl.BlockSpec((1, 512), lambda n: (0, 0)),
            pl.BlockSpec((512, 1), lambda n: (0, 0)),
            pl.BlockSpec((1, 1), lambda n: (0, 0)),
        ],
        out_specs=pl.BlockSpec((NG, 1), lambda n: (0, 0)),
        out_shape=jax.ShapeDtypeStruct((NG, 1), jnp.float32),
        scratch_shapes=[pltpu.VMEM((NG, CF + LANES), jnp.float32)],
    )(h3, batch3, w_h1, b_h1, w_h2, b_h2)


def kernel(x, edge_index, edge_attr, batch, W_vert, W_edge, W_elin, W_conv,
           b_conv, W_res, W_h1, b_h1, W_h2, b_h2):
    src = edge_index[0]
    dst = edge_index[1]

    h3 = _tc_init(x, W_vert)
    e3 = _tc_edges(edge_attr, W_edge, W_elin)

    for i in range(NL):
        ci_c = CIN[i] // LANES
        aggr_flat = _make_sc_aggr(ci_c)(
            src, dst,
            h3.reshape(ci_c * N_NODES, LANES),
            e3[i].reshape(ci_c * N_EDGES, LANES))
        aggr3 = aggr_flat.reshape(ci_c, N_PAD, LANES)
        h3 = _TC_NODE[i](h3, aggr3, W_conv[i],
                         b_conv[i].reshape(1, COUT[i]), W_res[i])

    batch3 = batch.reshape(G_POOL, 1, BN_POOL)
    return _tc_pool_head(h3, batch3, W_h1, b_h1.reshape(1, 512),
                         W_h2, b_h2.reshape(1, 1))


# speedup vs baseline: 1.8018x; 1.3320x over previous
import functools
import math

import jax
import jax.numpy as jnp
from jax import lax
from jax.experimental import pallas as pl
from jax.experimental.pallas import tpu as pltpu
from jax.experimental.pallas import tpu_sc as plsc

N_NODES = 50000
N_EDGES = 800000
NG = 64
H = 32
NL = 5
WIDTHS = [1, 1, 2, 3, 5, 7]
CIN = [w * H for w in WIDTHS[:-1]]
COUT = [w * H for w in WIDTHS[1:]]
INV_SQRT2 = 1.0 / math.sqrt(2.0)
EPS1 = 1.0 + 1e-05

NC = 2
NS = 16
LANES = 16

EPT = N_EDGES // NS
BE_SC = 2000
NB_SC = EPT // BE_SC
GCH = 80
NGC = BE_SC // GCH
N_PAD = 50048
RPT = N_PAD // NS
ZROWS = 184


@functools.lru_cache(maxsize=None)
def _make_sc_aggr(C):
    mesh = plsc.VectorSubcoreMesh(core_axis_name="c", subcore_axis_name="s")

    @functools.partial(
        pl.kernel,
        out_type=jax.ShapeDtypeStruct((C * N_PAD, LANES), jnp.float32),
        mesh=mesh,
        scratch_types=[
            pltpu.VMEM((NGC, GCH), jnp.int32),
            pltpu.VMEM((NGC, GCH), jnp.int32),
            pltpu.VMEM((NGC, GCH), jnp.int32),
            pltpu.VMEM((BE_SC, LANES), jnp.float32),
            pltpu.VMEM((BE_SC, LANES), jnp.float32),
            pltpu.VMEM((ZROWS, LANES), jnp.float32),
            pltpu.VMEM_SHARED((N_PAD, LANES), jnp.float32),
            pltpu.SemaphoreType.DMA,
            pltpu.SemaphoreType.DMA,
        ],
        compiler_params=pltpu.CompilerParams(use_tc_tiling_on_sc=False),
    )
    def sc_aggr(src_hbm, dst_hbm, h_hbm, e_hbm, out_hbm,
                srcv, dstv, idxv, hrows, erows, zrows, acc, sem, sem2):
        core = lax.axis_index("c")
        tile = lax.axis_index("s")

        def zfill(r, carry):
            zrows[r] = jnp.zeros((LANES,), jnp.float32)
            return carry
        lax.fori_loop(0, ZROWS, zfill, 0)

        def chunk_body(j, carry):
            c = j * NC + core

            for k in range(RPT // ZROWS):
                pltpu.sync_copy(zrows, acc.at[pl.ds(tile * RPT + k * ZROWS, ZROWS)])
            plsc.subcore_barrier()

            def batch_body(b, inner):
                e0 = tile * EPT + b * BE_SC
                r0 = tile * (EPT // GCH) + b * NGC
                d_src = pltpu.async_copy(src_hbm.at[pl.ds(r0, NGC)], srcv, sem2)
                d_dst = pltpu.async_copy(dst_hbm.at[pl.ds(r0, NGC)], dstv, sem2)
                d_e = pltpu.async_copy(e_hbm.at[pl.ds(c * N_EDGES + e0, BE_SC)],
                                       erows, sem2)
                d_src.wait()

                cbase = jnp.full((LANES,), c * N_NODES, jnp.int32)

                def idx_body(g, carry2):
                    for k in range(GCH // LANES):
                        sl = pl.ds(k * LANES, LANES)
                        idxv[g, sl] = srcv[g, sl] + cbase
                    return carry2
                lax.fori_loop(0, NGC, idx_body, 0)

                descs = [
                    pltpu.async_copy(h_hbm.at[idxv.at[g]],
                                     hrows.at[pl.ds(g * GCH, GCH)], sem)
                    for g in range(NGC)
                ]
                d_e.wait()
                d_dst.wait()
                for d in descs:
                    d.wait()

                def relu_body(r, carry2):
                    for u in range(8):
                        row = r * 8 + u
                        erows[row] = jnp.maximum(hrows[row] + erows[row], 0.0)
                    return carry2
                lax.fori_loop(0, BE_SC // 8, relu_body, 0)

                sdescs = [
                    pltpu.async_copy(erows.at[pl.ds(g * GCH, GCH)],
                                     acc.at[dstv.at[g]], sem2, add=True)
                    for g in range(NGC)
                ]
                for d in sdescs:
                    d.wait()
                return inner
            lax.fori_loop(0, NB_SC, batch_body, 0)
            plsc.subcore_barrier()

            pltpu.sync_copy(acc.at[pl.ds(tile * RPT, RPT)],
                            out_hbm.at[pl.ds(c * N_PAD + tile * RPT, RPT)])
            plsc.subcore_barrier()
            return carry
        lax.fori_loop(0, C // NC, chunk_body, 0)

    return sc_aggr


BN_INIT = 2000


def _tc_init_body(x_ref, wv_ref, out_ref):
    h = jnp.dot(x_ref[...], wv_ref[...], preferred_element_type=jnp.float32)
    for cc in range(H // LANES):
        out_ref[cc] = h[:, cc * LANES:(cc + 1) * LANES]


def _tc_init(x, w_vert):
    return pl.pallas_call(
        _tc_init_body,
        grid=(N_NODES // BN_INIT,),
        in_specs=[
            pl.BlockSpec((BN_INIT, 13), lambda n: (n, 0)),
            pl.BlockSpec((13, H), lambda n: (0, 0)),
        ],
        out_specs=pl.BlockSpec((H // LANES, BN_INIT, LANES), lambda n: (0, n, 0)),
        out_shape=jax.ShapeDtypeStruct((H // LANES, N_NODES, LANES), jnp.float32),
    )(x, w_vert)


BE_TC = 2000


def _tc_edge_body(ea_ref, we_ref, *rest):
    elin_refs = rest[:NL]
    out_refs = rest[NL:]
    t = jnp.dot(ea_ref[...], we_ref[...], preferred_element_type=jnp.float32)
    for i in range(NL):
        f = jnp.dot(t, elin_refs[i][...], preferred_element_type=jnp.float32)
        for cc in range(CIN[i] // LANES):
            out_refs[i][cc] = f[:, cc * LANES:(cc + 1) * LANES]


def _tc_edges(edge_attr, w_edge, w_elin):
    return pl.pallas_call(
        _tc_edge_body,
        grid=(N_EDGES // BE_TC,),
        in_specs=[
            pl.BlockSpec((BE_TC, 4), lambda n: (n, 0)),
            pl.BlockSpec((4, H), lambda n: (0, 0)),
        ] + [
            pl.BlockSpec((H, CIN[i]), lambda n: (0, 0)) for i in range(NL)
        ],
        out_specs=[
            pl.BlockSpec((CIN[i] // LANES, BE_TC, LANES), lambda n: (0, n, 0))
            for i in range(NL)
        ],
        out_shape=[
            jax.ShapeDtypeStruct((CIN[i] // LANES, N_EDGES, LANES), jnp.float32)
            for i in range(NL)
        ],
    )(edge_attr, w_edge, *w_elin)


BN_NODE = 1000


def _make_tc_node(i):
    ci_c = CIN[i] // LANES
    co_c = COUT[i] // LANES

    def body(h_ref, a_ref, wc_ref, bc_ref, wr_ref, out_ref):
        hcat = jnp.concatenate([h_ref[cc] for cc in range(ci_c)], axis=1)
        acat = jnp.concatenate([a_ref[cc] for cc in range(ci_c)], axis=1)
        z = EPS1 * hcat + acat
        out = jnp.maximum(
            jnp.dot(z, wc_ref[...], preferred_element_type=jnp.float32) + bc_ref[...],
            0.0)
        hn = (out + jnp.dot(hcat, wr_ref[...], preferred_element_type=jnp.float32)
              ) * INV_SQRT2
        for cc in range(co_c):
            out_ref[cc] = hn[:, cc * LANES:(cc + 1) * LANES]

    def run(h3, aggr3, w_conv, b_conv, w_res):
        return pl.pallas_call(
            body,
            grid=(N_NODES // BN_NODE,),
            in_specs=[
                pl.BlockSpec((ci_c, BN_NODE, LANES), lambda n: (0, n, 0)),
                pl.BlockSpec((ci_c, BN_NODE, LANES), lambda n: (0, n, 0)),
                pl.BlockSpec((CIN[i], COUT[i]), lambda n: (0, 0)),
                pl.BlockSpec((1, COUT[i]), lambda n: (0, 0)),
                pl.BlockSpec((CIN[i], COUT[i]), lambda n: (0, 0)),
            ],
            out_specs=pl.BlockSpec((co_c, BN_NODE, LANES), lambda n: (0, n, 0)),
            out_shape=jax.ShapeDtypeStruct((co_c, N_NODES, LANES), jnp.float32),
        )(h3, aggr3, w_conv, b_conv, w_res)
    return run


_TC_NODE = [_make_tc_node(i) for i in range(NL)]

BN_POOL = 1000
G_POOL = N_NODES // BN_POOL
CF = COUT[-1]
CF_C = CF // LANES


def _erf(x):
    a1, a2, a3, a4, a5 = (0.254829592, -0.284496736, 1.421413741,
                          -1.453152027, 1.061405429)
    p = 0.3275911
    s = jnp.sign(x)
    ax = jnp.abs(x)
    t = 1.0 / (1.0 + p * ax)
    poly = ((((a5 * t + a4) * t + a3) * t + a2) * t + a1) * t
    return s * (1.0 - poly * jnp.exp(-ax * ax))


def _tc_pool_head_body(h_ref, b_ref, wh1_ref, bh1_ref, wh2_ref, bh2_ref,
                       out_ref, acc_ref):
    n = pl.program_id(0)
    bid = b_ref[0]
    onehot_t = (lax.broadcasted_iota(jnp.int32, (NG, BN_POOL), 0)
                == jnp.broadcast_to(bid, (NG, BN_POOL))).astype(jnp.float32)
    hcat = jnp.concatenate(
        [h_ref[cc] for cc in range(CF_C)]
        + [jnp.ones((BN_POOL, LANES), jnp.float32)], axis=1)
    part = jnp.dot(onehot_t, hcat, preferred_element_type=jnp.float32)

    @pl.when(n == 0)
    def _init():
        acc_ref[...] = part

    @pl.when(n > 0)
    def _accum():
        acc_ref[...] = acc_ref[...] + part

    @pl.when(n == G_POOL - 1)
    def _head():
        sums = acc_ref[:, :CF]
        cnt = jnp.maximum(acc_ref[:, CF:CF + 1], 1.0)
        pooled = sums / cnt
        g1 = jnp.dot(pooled, wh1_ref[...],
                     preferred_element_type=jnp.float32) + bh1_ref[...]
        ge = g1 * 0.5 * (1.0 + _erf(g1 * INV_SQRT2))
        out_ref[...] = jnp.dot(ge, wh2_ref[...],
                               preferred_element_type=jnp.float32) + bh2_ref[...]


def _tc_pool_head(h3, batch3, w_h1, b_h1, w_h2, b_h2):
    return pl.pallas_call(
        _tc_pool_head_body,
        grid=(G_POOL,),
        in_specs=[
            pl.BlockSpec((CF_C, BN_POOL, LANES), lambda n: (0, n, 0)),
            pl.BlockSpec((1, 1, BN_POOL), lambda n: (n, 0, 0)),
            pl.BlockSpec((CF, 512), lambda n: (0, 0)),
            pl.BlockSpec((1, 512), lambda n: (0, 0)),
            pl.BlockSpec((512, 1), lambda n: (0, 0)),
            pl.BlockSpec((1, 1), lambda n: (0, 0)),
        ],
        out_specs=pl.BlockSpec((NG, 1), lambda n: (0, 0)),
        out_shape=jax.ShapeDtypeStruct((NG, 1), jnp.float32),
        scratch_shapes=[pltpu.VMEM((NG, CF + LANES), jnp.float32)],
    )(h3, batch3, w_h1, b_h1, w_h2, b_h2)


def kernel(x, edge_index, edge_attr, batch, W_vert, W_edge, W_elin, W_conv,
           b_conv, W_res, W_h1, b_h1, W_h2, b_h2):
    src = edge_index[0].reshape(N_EDGES // GCH, GCH)
    dst = edge_index[1].reshape(N_EDGES // GCH, GCH)

    h3 = _tc_init(x, W_vert)
    e3 = _tc_edges(edge_attr, W_edge, W_elin)

    for i in range(NL):
        ci_c = CIN[i] // LANES
        aggr_flat = _make_sc_aggr(ci_c)(
            src, dst,
            h3.reshape(ci_c * N_NODES, LANES),
            e3[i].reshape(ci_c * N_EDGES, LANES))
        aggr3 = aggr_flat.reshape(ci_c, N_PAD, LANES)
        h3 = _TC_NODE[i](h3, aggr3, W_conv[i],
                         b_conv[i].reshape(1, COUT[i]), W_res[i])

    batch3 = batch.reshape(G_POOL, 1, BN_POOL)
    return _tc_pool_head(h3, batch3, W_h1, b_h1.reshape(1, 512),
                         W_h2, b_h2.reshape(1, 1))


# speedup vs baseline: 1.8323x; 1.0169x over previous
import functools
import math

import jax
import jax.numpy as jnp
from jax import lax
from jax.experimental import pallas as pl
from jax.experimental.pallas import tpu as pltpu
from jax.experimental.pallas import tpu_sc as plsc

N_NODES = 50000
N_EDGES = 800000
NG = 64
H = 32
NL = 5
WIDTHS = [1, 1, 2, 3, 5, 7]
CIN = [w * H for w in WIDTHS[:-1]]
COUT = [w * H for w in WIDTHS[1:]]
INV_SQRT2 = 1.0 / math.sqrt(2.0)
EPS1 = 1.0 + 1e-05

NC = 2
NS = 16
LANES = 16

EPT = N_EDGES // NS
BE_SC = 2000
NB_SC = EPT // BE_SC
GCH = 80
NGC = BE_SC // GCH
N_PAD = 50048
RPT = N_PAD // NS
ZROWS = 184


@functools.lru_cache(maxsize=None)
def _make_sc_aggr(C):
    mesh = plsc.VectorSubcoreMesh(core_axis_name="c", subcore_axis_name="s")

    @functools.partial(
        pl.kernel,
        out_type=jax.ShapeDtypeStruct((C * N_PAD, LANES), jnp.float32),
        mesh=mesh,
        scratch_types=[
            pltpu.VMEM((NGC, GCH), jnp.int32),
            pltpu.VMEM((NGC, GCH), jnp.int32),
            pltpu.VMEM((NGC, GCH), jnp.int32),
            pltpu.VMEM((BE_SC, LANES), jnp.float32),
            pltpu.VMEM((BE_SC, LANES), jnp.float32),
            pltpu.VMEM((ZROWS, LANES), jnp.float32),
            pltpu.VMEM_SHARED((N_PAD, LANES), jnp.float32),
            pltpu.SemaphoreType.DMA,
            pltpu.SemaphoreType.DMA,
        ],
        compiler_params=pltpu.CompilerParams(use_tc_tiling_on_sc=False),
    )
    def sc_aggr(src_hbm, dst_hbm, h_hbm, e_hbm, out_hbm,
                srcv, dstv, idxv, hrows, erows, zrows, acc, sem, sem2):
        core = lax.axis_index("c")
        tile = lax.axis_index("s")

        def zfill(r, carry):
            zrows[r] = jnp.zeros((LANES,), jnp.float32)
            return carry
        lax.fori_loop(0, ZROWS, zfill, 0)

        def chunk_body(j, carry):
            c = j * NC + core

            for k in range(RPT // ZROWS):
                pltpu.sync_copy(zrows, acc.at[pl.ds(tile * RPT + k * ZROWS, ZROWS)])
            plsc.subcore_barrier()

            def batch_body(b, inner):
                e0 = tile * EPT + b * BE_SC
                r0 = tile * (EPT // GCH) + b * NGC
                d_src = pltpu.async_copy(src_hbm.at[pl.ds(r0, NGC)], srcv, sem2)
                d_dst = pltpu.async_copy(dst_hbm.at[pl.ds(r0, NGC)], dstv, sem2)
                d_e = pltpu.async_copy(e_hbm.at[pl.ds(c * N_EDGES + e0, BE_SC)],
                                       erows, sem2)
                d_src.wait()

                cbase = jnp.full((LANES,), c * N_NODES, jnp.int32)

                def idx_body(g, carry2):
                    for k in range(GCH // LANES):
                        sl = pl.ds(k * LANES, LANES)
                        idxv[g, sl] = srcv[g, sl] + cbase
                    return carry2
                lax.fori_loop(0, NGC, idx_body, 0)

                descs = [
                    pltpu.async_copy(h_hbm.at[idxv.at[g]],
                                     hrows.at[pl.ds(g * GCH, GCH)], sem)
                    for g in range(NGC)
                ]
                d_e.wait()
                d_dst.wait()

                sdescs = []
                for g in range(NGC):
                    descs[g].wait()

                    def relu_body(r, carry2, _g=g):
                        for u in range(8):
                            row = _g * GCH + r * 8 + u
                            erows[row] = jnp.maximum(hrows[row] + erows[row], 0.0)
                        return carry2
                    lax.fori_loop(0, GCH // 8, relu_body, 0)
                    sdescs.append(
                        pltpu.async_copy(erows.at[pl.ds(g * GCH, GCH)],
                                         acc.at[dstv.at[g]], sem2, add=True))
                for d in sdescs:
                    d.wait()
                return inner
            lax.fori_loop(0, NB_SC, batch_body, 0)
            plsc.subcore_barrier()

            pltpu.sync_copy(acc.at[pl.ds(tile * RPT, RPT)],
                            out_hbm.at[pl.ds(c * N_PAD + tile * RPT, RPT)])
            plsc.subcore_barrier()
            return carry
        lax.fori_loop(0, C // NC, chunk_body, 0)

    return sc_aggr


BN_INIT = 2000


def _tc_init_body(x_ref, wv_ref, out_ref):
    h = jnp.dot(x_ref[...], wv_ref[...], preferred_element_type=jnp.float32)
    for cc in range(H // LANES):
        out_ref[cc] = h[:, cc * LANES:(cc + 1) * LANES]


def _tc_init(x, w_vert):
    return pl.pallas_call(
        _tc_init_body,
        grid=(N_NODES // BN_INIT,),
        in_specs=[
            pl.BlockSpec((BN_INIT, 13), lambda n: (n, 0)),
            pl.BlockSpec((13, H), lambda n: (0, 0)),
        ],
        out_specs=pl.BlockSpec((H // LANES, BN_INIT, LANES), lambda n: (0, n, 0)),
        out_shape=jax.ShapeDtypeStruct((H // LANES, N_NODES, LANES), jnp.float32),
    )(x, w_vert)


BE_TC = 2000


def _tc_edge_body(ea_ref, we_ref, *rest):
    elin_refs = rest[:NL]
    out_refs = rest[NL:]
    t = jnp.dot(ea_ref[...], we_ref[...], preferred_element_type=jnp.float32)
    for i in range(NL):
        f = jnp.dot(t, elin_refs[i][...], preferred_element_type=jnp.float32)
        for cc in range(CIN[i] // LANES):
            out_refs[i][cc] = f[:, cc * LANES:(cc + 1) * LANES]


def _tc_edges(edge_attr, w_edge, w_elin):
    return pl.pallas_call(
        _tc_edge_body,
        grid=(N_EDGES // BE_TC,),
        in_specs=[
            pl.BlockSpec((BE_TC, 4), lambda n: (n, 0)),
            pl.BlockSpec((4, H), lambda n: (0, 0)),
        ] + [
            pl.BlockSpec((H, CIN[i]), lambda n: (0, 0)) for i in range(NL)
        ],
        out_specs=[
            pl.BlockSpec((CIN[i] // LANES, BE_TC, LANES), lambda n: (0, n, 0))
            for i in range(NL)
        ],
        out_shape=[
            jax.ShapeDtypeStruct((CIN[i] // LANES, N_EDGES, LANES), jnp.float32)
            for i in range(NL)
        ],
    )(edge_attr, w_edge, *w_elin)


BN_NODE = 1000


def _make_tc_node(i):
    ci_c = CIN[i] // LANES
    co_c = COUT[i] // LANES

    def body(h_ref, a_ref, wc_ref, bc_ref, wr_ref, out_ref):
        hcat = jnp.concatenate([h_ref[cc] for cc in range(ci_c)], axis=1)
        acat = jnp.concatenate([a_ref[cc] for cc in range(ci_c)], axis=1)
        z = EPS1 * hcat + acat
        out = jnp.maximum(
            jnp.dot(z, wc_ref[...], preferred_element_type=jnp.float32) + bc_ref[...],
            0.0)
        hn = (out + jnp.dot(hcat, wr_ref[...], preferred_element_type=jnp.float32)
              ) * INV_SQRT2
        for cc in range(co_c):
            out_ref[cc] = hn[:, cc * LANES:(cc + 1) * LANES]

    def run(h3, aggr3, w_conv, b_conv, w_res):
        return pl.pallas_call(
            body,
            grid=(N_NODES // BN_NODE,),
            in_specs=[
                pl.BlockSpec((ci_c, BN_NODE, LANES), lambda n: (0, n, 0)),
                pl.BlockSpec((ci_c, BN_NODE, LANES), lambda n: (0, n, 0)),
                pl.BlockSpec((CIN[i], COUT[i]), lambda n: (0, 0)),
                pl.BlockSpec((1, COUT[i]), lambda n: (0, 0)),
                pl.BlockSpec((CIN[i], COUT[i]), lambda n: (0, 0)),
            ],
            out_specs=pl.BlockSpec((co_c, BN_NODE, LANES), lambda n: (0, n, 0)),
            out_shape=jax.ShapeDtypeStruct((co_c, N_NODES, LANES), jnp.float32),
        )(h3, aggr3, w_conv, b_conv, w_res)
    return run


_TC_NODE = [_make_tc_node(i) for i in range(NL)]

BN_POOL = 1000
G_POOL = N_NODES // BN_POOL
CF = COUT[-1]
CF_C = CF // LANES


def _erf(x):
    a1, a2, a3, a4, a5 = (0.254829592, -0.284496736, 1.421413741,
                          -1.453152027, 1.061405429)
    p = 0.3275911
    s = jnp.sign(x)
    ax = jnp.abs(x)
    t = 1.0 / (1.0 + p * ax)
    poly = ((((a5 * t + a4) * t + a3) * t + a2) * t + a1) * t
    return s * (1.0 - poly * jnp.exp(-ax * ax))


def _tc_pool_head_body(h_ref, b_ref, wh1_ref, bh1_ref, wh2_ref, bh2_ref,
                       out_ref, acc_ref):
    n = pl.program_id(0)
    bid = b_ref[0]
    onehot_t = (lax.broadcasted_iota(jnp.int32, (NG, BN_POOL), 0)
                == jnp.broadcast_to(bid, (NG, BN_POOL))).astype(jnp.float32)
    hcat = jnp.concatenate(
        [h_ref[cc] for cc in range(CF_C)]
        + [jnp.ones((BN_POOL, LANES), jnp.float32)], axis=1)
    part = jnp.dot(onehot_t, hcat, preferred_element_type=jnp.float32)

    @pl.when(n == 0)
    def _init():
        acc_ref[...] = part

    @pl.when(n > 0)
    def _accum():
        acc_ref[...] = acc_ref[...] + part

    @pl.when(n == G_POOL - 1)
    def _head():
        sums = acc_ref[:, :CF]
        cnt = jnp.maximum(acc_ref[:, CF:CF + 1], 1.0)
        pooled = sums / cnt
        g1 = jnp.dot(pooled, wh1_ref[...],
                     preferred_element_type=jnp.float32) + bh1_ref[...]
        ge = g1 * 0.5 * (1.0 + _erf(g1 * INV_SQRT2))
        out_ref[...] = jnp.dot(ge, wh2_ref[...],
                               preferred_element_type=jnp.float32) + bh2_ref[...]


def _tc_pool_head(h3, batch3, w_h1, b_h1, w_h2, b_h2):
    return pl.pallas_call(
        _tc_pool_head_body,
        grid=(G_POOL,),
        in_specs=[
            pl.BlockSpec((CF_C, BN_POOL, LANES), lambda n: (0, n, 0)),
            pl.BlockSpec((1, 1, BN_POOL), lambda n: (n, 0, 0)),
            pl.BlockSpec((CF, 512), lambda n: (0, 0)),
            pl.BlockSpec((1, 512), lambda n: (0, 0)),
            pl.BlockSpec((512, 1), lambda n: (0, 0)),
            pl.BlockSpec((1, 1), lambda n: (0, 0)),
        ],
        out_specs=pl.BlockSpec((NG, 1), lambda n: (0, 0)),
        out_shape=jax.ShapeDtypeStruct((NG, 1), jnp.float32),
        scratch_shapes=[pltpu.VMEM((NG, CF + LANES), jnp.float32)],
    )(h3, batch3, w_h1, b_h1, w_h2, b_h2)


def kernel(x, edge_index, edge_attr, batch, W_vert, W_edge, W_elin, W_conv,
           b_conv, W_res, W_h1, b_h1, W_h2, b_h2):
    src = edge_index[0].reshape(N_EDGES // GCH, GCH)
    dst = edge_index[1].reshape(N_EDGES // GCH, GCH)

    h3 = _tc_init(x, W_vert)
    e3 = _tc_edges(edge_attr, W_edge, W_elin)

    for i in range(NL):
        ci_c = CIN[i] // LANES
        aggr_flat = _make_sc_aggr(ci_c)(
            src, dst,
            h3.reshape(ci_c * N_NODES, LANES),
            e3[i].reshape(ci_c * N_EDGES, LANES))
        aggr3 = aggr_flat.reshape(ci_c, N_PAD, LANES)
        h3 = _TC_NODE[i](h3, aggr3, W_conv[i],
                         b_conv[i].reshape(1, COUT[i]), W_res[i])

    batch3 = batch.reshape(G_POOL, 1, BN_POOL)
    return _tc_pool_head(h3, batch3, W_h1, b_h1.reshape(1, 512),
                         W_h2, b_h2.reshape(1, 1))
